# baseline (speedup 1.0000x reference)
import functools
import jax
import jax.numpy as jnp
from jax import lax
from jax.experimental import pallas as pl
from jax.experimental.pallas import tpu as pltpu
from jax.experimental.pallas import tpu_sc as plsc

N = 50000
D = 256


_MM_BLK = 1000


def _mm_body(x_ref, w_ref, b_ref, o_ref):
    acc = jnp.dot(x_ref[...], w_ref[...], preferred_element_type=jnp.float32)
    o_ref[...] = jnp.maximum(acc + b_ref[...], 0.0)


def _matmul_relu(x, w, b2):
    return pl.pallas_call(
        _mm_body,
        grid=(N // _MM_BLK,),
        in_specs=[
            pl.BlockSpec((_MM_BLK, D), lambda i: (i, 0)),
            pl.BlockSpec((D, D), lambda i: (0, 0)),
            pl.BlockSpec((1, D), lambda i: (0, 0)),
        ],
        out_specs=pl.BlockSpec((_MM_BLK, D), lambda i: (i, 0)),
        out_shape=jax.ShapeDtypeStruct((N, D), jnp.float32),
    )(x, w, b2)


NE_PAD = 50176
CHUNK = NE_PAD // 16
GROUPS = CHUNK // 16
TR = 240
R_SC = 16 * TR
PASSES = 7
LB = 3584
FB = 4224
DMA_B = 128


def _sc_body(h_hbm, adj_hbm, out_hbm,
             adj_t, srcid, dstv, fsrc, fdst, stage, acc,
             cntbuf, cnt_local, shsrc, shdst, shcnt, sem):
    c = lax.axis_index("c")
    s = lax.axis_index("s")
    iota = lax.iota(jnp.int32, 16)
    zf = jnp.zeros((16,), jnp.float32)

    pltpu.sync_copy(adj_hbm.at[pl.ds(s * CHUNK, CHUNK)], adj_t)

    def _pass(p, carry):
        lo = p * (2 * R_SC) + c * R_SC

        def _filt(g, cursor):
            idxv = adj_t[pl.ds(g * 16, 16)]
            m = (idxv >= lo) & (idxv < lo + R_SC)
            mi = jnp.where(m, 1, 0).astype(jnp.int32)
            incl = plsc.cumsum(mi)
            pos = cursor + incl - 1
            eid = s * CHUNK + g * 16 + iota
            plsc.store_scatter(srcid, [pos], eid, mask=m)
            plsc.store_scatter(dstv, [pos], idxv - lo, mask=m)
            return cursor + incl[15]

        n1 = lax.fori_loop(0, GROUPS, _filt, jnp.int32(0))

        def _seg(k, carry2):
            pltpu.sync_copy(srcid.at[pl.ds(k * 512, 512)],
                            shsrc.at[s].at[pl.ds(k * 512, 512)])
            pltpu.sync_copy(dstv.at[pl.ds(k * 512, 512)],
                            shdst.at[s].at[pl.ds(k * 512, 512)])
            return carry2

        lax.fori_loop(0, (n1 + 511) >> 9, _seg, 0)
        cntbuf[pl.ds(0, 16)] = jnp.full((16,), n1, jnp.int32)
        cntbuf[pl.ds(16, 16)] = jnp.full((16,), n1, jnp.int32)
        pltpu.sync_copy(cntbuf, shcnt.at[s])
        plsc.subcore_barrier()

        olo = s * TR

        def _zr(r, carry2):
            for k in range(D // 16):
                acc[r, pl.ds(k * 16, 16)] = zf
            return carry2

        lax.fori_loop(0, TR, _zr, 0)
        pltpu.sync_copy(shcnt, cnt_local)

        def _flush_chunk(q, carry2):
            cp = pltpu.make_async_copy(
                h_hbm.at[fsrc.at[pl.ds(q * DMA_B, DMA_B)]], stage, sem)
            cp.start()
            cp.wait()

            def _grp(g, carry3):
                dvec = fdst[pl.ds(q * DMA_B + g * 16, 16)]
                for l in range(16):
                    o_r = dvec[l]
                    for k in range(D // 16):
                        acc[o_r, pl.ds(k * 16, 16)] += (
                            stage[g * 16 + l, pl.ds(k * 16, 16)])
                return carry3

            lax.fori_loop(0, 8, _grp, 0)
            return carry2

        def _src_t(t, cursor):
            nt = cnt_local[t, pl.ds(0, 16)][0]

            def _seg2(k, carry2):
                pltpu.sync_copy(shsrc.at[t].at[pl.ds(k * 512, 512)],
                                srcid.at[pl.ds(k * 512, 512)])
                pltpu.sync_copy(shdst.at[t].at[pl.ds(k * 512, 512)],
                                dstv.at[pl.ds(k * 512, 512)])
                return carry2

            lax.fori_loop(0, (nt + 511) >> 9, _seg2, 0)

            def _scan(g, cur):
                sv = srcid[pl.ds(g * 16, 16)]
                dv = dstv[pl.ds(g * 16, 16)]
                valid = (g * 16 + iota) < nt
                m = valid & (dv >= olo) & (dv < olo + TR)
                mi = jnp.where(m, 1, 0).astype(jnp.int32)
                incl = plsc.cumsum(mi)
                pos = cur + incl - 1
                plsc.store_scatter(fsrc, [pos], sv, mask=m)
                plsc.store_scatter(fdst, [pos], dv - olo, mask=m)
                return cur + incl[15]

            cursor = lax.fori_loop(0, (nt + 15) >> 4, _scan, cursor)

            nch = cursor >> 7
            lax.fori_loop(0, nch, _flush_chunk, 0)
            base = nch * DMA_B
            for g in range(8):
                sv = fsrc[pl.ds(base + g * 16, 16)]
                dv = fdst[pl.ds(base + g * 16, 16)]
                fsrc[pl.ds(g * 16, 16)] = sv
                fdst[pl.ds(g * 16, 16)] = dv
            return cursor - nch * DMA_B

        cursor = lax.fori_loop(0, 16, _src_t, jnp.int32(0))

        @pl.when(cursor > 0)
        def _():
            for k in range(8):
                pos = cursor + k * 16 + iota
                plsc.store_scatter(fsrc, [pos], jnp.zeros((16,), jnp.int32))
                plsc.store_scatter(fdst, [pos], jnp.full((16,), TR, jnp.int32))
            _flush_chunk(0, 0)

        obase = lo + olo

        @pl.when(obase + TR <= N)
        def _():
            pltpu.sync_copy(acc.at[pl.ds(0, TR)],
                            out_hbm.at[pl.ds(obase, TR)])

        @pl.when((obase + TR > N) & (obase < N))
        def _():
            for sb in range(TR // 16):
                @pl.when(obase + (sb + 1) * 16 <= N)
                def _():
                    pltpu.sync_copy(
                        acc.at[pl.ds(sb * 16, 16)],
                        out_hbm.at[pl.ds(obase + sb * 16, 16)])

        plsc.subcore_barrier()
        return carry

    lax.fori_loop(0, PASSES, _pass, 0)


@functools.cache
def _get_sc_call():
    return pl.kernel(
        _sc_body,
        out_type=jax.ShapeDtypeStruct((N, D), jnp.float32),
        mesh=plsc.VectorSubcoreMesh(core_axis_name="c", subcore_axis_name="s"),
        scratch_types=[
            pltpu.VMEM((CHUNK,), jnp.int32),
            pltpu.VMEM((LB,), jnp.int32),
            pltpu.VMEM((LB,), jnp.int32),
            pltpu.VMEM((FB,), jnp.int32),
            pltpu.VMEM((FB,), jnp.int32),
            pltpu.VMEM((DMA_B, D), jnp.float32),
            pltpu.VMEM((TR + 8, D), jnp.float32),
            pltpu.VMEM((128,), jnp.int32),
            pltpu.VMEM((16, 128), jnp.int32),
            pltpu.VMEM_SHARED((16, LB), jnp.int32),
            pltpu.VMEM_SHARED((16, LB), jnp.int32),
            pltpu.VMEM_SHARED((16, 128), jnp.int32),
            pltpu.SemaphoreType.DMA,
        ],
        compiler_params=pltpu.CompilerParams(needs_layout_passes=False),
    )


@jax.jit
def kernel(x, adj, w, b):
    h = _matmul_relu(x, w, b.reshape(1, D))
    adj32 = adj.astype(jnp.int32)
    adj_p = jnp.full((NE_PAD,), -1, jnp.int32).at[:N].set(adj32)
    return _get_sc_call()(h, adj_p)
